# Initial kernel scaffold; baseline (speedup 1.0000x reference)
import jax
import jax.numpy as jnp
from jax.experimental import pallas as pl


def kernel(point_patches, nn_idx, center_number, W1a, b1a, g1, be1, W1b, b1b, W2a, b2a, g2, be2, W2b, b2b):
    raise NotImplementedError("write your pallas kernel here")


# speedup vs baseline: 1.0073x; 1.0073x over previous
import functools

import jax
import jax.numpy as jnp
from jax import lax
from jax.experimental import pallas as pl
from jax.experimental.pallas import tpu as pltpu

B, N, CENTER = 8, 16384, 1024
IN_C, H0, H1, OUT_C = 32, 64, 64, 64

TILE1 = 2048


def _ln(x, g, b):
    mu = jnp.mean(x, axis=-1, keepdims=True)
    var = jnp.mean((x - mu) ** 2, axis=-1, keepdims=True)
    return (x - mu) * lax.rsqrt(var + 1e-5) * g + b


def _gelu(x):
    return 0.5 * x * (1.0 + lax.erf(x * 0.7071067811865476))


def _mlp1_body(pts_ref, W1a_ref, b1a_ref, g1_ref, be1_ref, W1b_ref, b1b_ref,
               out_ref):
    x = pts_ref[...]
    x = jnp.dot(x, W1a_ref[...], preferred_element_type=jnp.float32)
    x = x + b1a_ref[...]
    x = _ln(x, g1_ref[...], be1_ref[...])
    x = _gelu(x)
    x = jnp.dot(x, W1b_ref[...], preferred_element_type=jnp.float32)
    out_ref[...] = x + b1b_ref[...]


def _mlp1(points_flat, W1a, b1a, g1, be1, W1b, b1b):
    n_rows = points_flat.shape[0]
    grid = (n_rows // TILE1,)
    return pl.pallas_call(
        _mlp1_body,
        grid=grid,
        in_specs=[
            pl.BlockSpec((TILE1, IN_C), lambda i: (i, 0)),
            pl.BlockSpec((IN_C, H0), lambda i: (0, 0)),
            pl.BlockSpec((H0,), lambda i: (0,)),
            pl.BlockSpec((H0,), lambda i: (0,)),
            pl.BlockSpec((H0,), lambda i: (0,)),
            pl.BlockSpec((H0, H0), lambda i: (0, 0)),
            pl.BlockSpec((H0,), lambda i: (0,)),
        ],
        out_specs=pl.BlockSpec((TILE1, H0), lambda i: (i, 0)),
        out_shape=jax.ShapeDtypeStruct((n_rows, H0), jnp.float32),
    )(points_flat, W1a, b1a, g1, be1, W1b, b1b)


def _mlp2_body(xcat_ref, W2a_ref, b2a_ref, g2_ref, be2_ref, W2b_ref, b2b_ref,
               out_ref):
    x = xcat_ref[...]
    x = jnp.dot(x, W2a_ref[...], preferred_element_type=jnp.float32)
    x = x + b2a_ref[...]
    x = _ln(x, g2_ref[...], be2_ref[...])
    x = _gelu(x)
    x = jnp.dot(x, W2b_ref[...], preferred_element_type=jnp.float32)
    out_ref[...] = x + b2b_ref[...]


def _mlp2(xcat, W2a, b2a, g2, be2, W2b, b2b):
    n_rows = xcat.shape[0]
    grid = (n_rows // TILE1,)
    return pl.pallas_call(
        _mlp2_body,
        grid=grid,
        in_specs=[
            pl.BlockSpec((TILE1, 2 * H0), lambda i: (i, 0)),
            pl.BlockSpec((2 * H0, H1), lambda i: (0, 0)),
            pl.BlockSpec((H1,), lambda i: (0,)),
            pl.BlockSpec((H1,), lambda i: (0,)),
            pl.BlockSpec((H1,), lambda i: (0,)),
            pl.BlockSpec((H1, OUT_C), lambda i: (0, 0)),
            pl.BlockSpec((OUT_C,), lambda i: (0,)),
        ],
        out_specs=pl.BlockSpec((TILE1, OUT_C), lambda i: (i, 0)),
        out_shape=jax.ShapeDtypeStruct((n_rows, OUT_C), jnp.float32),
    )(xcat, W2a, b2a, g2, be2, W2b, b2b)


def kernel(point_patches, nn_idx, center_number, W1a, b1a, g1, be1, W1b, b1b,
           W2a, b2a, g2, be2, W2b, b2b):
    pts = point_patches.reshape(B * N, IN_C)
    x1 = _mlp1(pts, W1a, b1a, g1, be1, W1b, b1b).reshape(B, N, H0)

    bidx = jnp.arange(B)[:, None]
    y1 = jnp.zeros((B, CENTER, H0), jnp.float32).at[bidx, nn_idx].max(x1)
    idx3 = jnp.broadcast_to(nn_idx[:, :, None], (B, N, H0))
    x_max = jnp.take_along_axis(y1, idx3, axis=1)
    xcat = jnp.concatenate([x_max, x1], axis=-1).reshape(B * N, 2 * H0)

    x2 = _mlp2(xcat, W2a, b2a, g2, be2, W2b, b2b).reshape(B, N, OUT_C)
    y2 = jnp.zeros((B, CENTER, OUT_C), jnp.float32).at[bidx, nn_idx].max(x2)
    return y2
